# baseline (speedup 1.0000x reference)
import functools

import jax
import jax.numpy as jnp
from jax import lax
from jax.experimental import pallas as pl
from jax.experimental.pallas import tpu as pltpu
from jax.experimental.pallas import tpu_sc as plsc

N = 10000
E = 320000
D = 128

NC = 2
NS = 16
NW = NC * NS
CHUNK = 80
EPT = E // NW
NCHUNK = EPT // CHUNK
RPT = 624
RREM = N - NS * RPT
BN_SCALE = float(1.0 / (1.0 + 1e-5) ** 0.5)

ECHUNK = 128
NGRP = 80
NBUF = 2
E_PAD = NW * NGRP * ECHUNK
N_SP = N + 8

_sc_mesh = plsc.VectorSubcoreMesh(core_axis_name="c", subcore_axis_name="s")


@functools.partial(
    pl.kernel,
    out_type=jax.ShapeDtypeStruct((NC, N), jnp.float32),
    mesh=_sc_mesh,
    scratch_types=[
        pltpu.VMEM_SHARED((N,), jnp.float32),
        pltpu.VMEM((CHUNK,), jnp.int32),
        pltpu.VMEM((CHUNK,), jnp.float32),
        pltpu.VMEM((N,), jnp.float32),
    ],
)
def _deg_sc(dst_hbm, hist_hbm, hist_sp, dst_v, ones_v, stage_v):
    c = lax.axis_index("c")
    s = lax.axis_index("s")
    wid = c * NS + s

    def fill_ones(i, _):
        ones_v[pl.ds(i * 16, 16)] = jnp.ones((16,), jnp.float32)
        return 0

    lax.fori_loop(0, CHUNK // 16, fill_ones, 0)

    @pl.when(s == 0)
    def _():
        def zero(i, _):
            stage_v[pl.ds(i * 16, 16)] = jnp.zeros((16,), jnp.float32)
            return 0

        lax.fori_loop(0, N // 16, zero, 0)
        pltpu.sync_copy(stage_v, hist_sp)

    plsc.subcore_barrier()

    def body(i, _):
        base = wid * EPT + i * CHUNK
        pltpu.sync_copy(dst_hbm.at[pl.ds(base, CHUNK)], dst_v)
        pltpu.sync_copy(ones_v, hist_sp.at[dst_v], add=True)
        return 0

    lax.fori_loop(0, NCHUNK, body, 0)
    plsc.subcore_barrier()

    @pl.when(s == 0)
    def _():
        pltpu.sync_copy(hist_sp, stage_v)
        pltpu.sync_copy(stage_v, hist_hbm.at[c])


@functools.partial(
    pl.kernel,
    out_type=jax.ShapeDtypeStruct((NC, N, D), jnp.float32),
    mesh=_sc_mesh,
    scratch_types=[
        pltpu.VMEM_SHARED((N_SP, D), jnp.float32),
        pltpu.VMEM((NGRP, ECHUNK), jnp.int32),
        pltpu.VMEM((NBUF, ECHUNK), jnp.int32),
        pltpu.VMEM((NBUF, ECHUNK), jnp.int32),
        pltpu.VMEM((NBUF, ECHUNK, D), jnp.float32),
        pltpu.SemaphoreType.DMA((NBUF,)),
    ],
)
def _agg_sc(h_hbm, idx_hbm, out_hbm, agg_sp, idx_v, src_u, dst_u, rows_v,
            sems):
    c = lax.axis_index("c")
    s = lax.axis_index("s")
    wid = c * NS + s

    pltpu.sync_copy(idx_hbm.at[wid], idx_v)
    r0 = s * RPT
    pltpu.sync_copy(h_hbm.at[pl.ds(r0, RPT)], agg_sp.at[pl.ds(r0, RPT)])

    @pl.when(s == NS - 1)
    def _():
        pltpu.sync_copy(h_hbm.at[pl.ds(NS * RPT, RREM)],
                        agg_sp.at[pl.ds(NS * RPT, RREM)])

    def unpack(ch, b):
        for k in range(ECHUNK // 16):
            w = idx_v[ch, pl.ds(k * 16, 16)]
            src_u[b, pl.ds(k * 16, 16)] = lax.shift_right_logical(w, 16)
            dst_u[b, pl.ds(k * 16, 16)] = lax.bitwise_and(w, 0xFFFF)

    plsc.subcore_barrier()

    for b in range(NBUF):
        unpack(b, b)
        pltpu.async_copy(h_hbm.at[src_u.at[b]], rows_v.at[b], sems.at[b])

    def group(g, _):
        for b in range(NBUF):
            ch = g * NBUF + b
            pltpu.make_async_copy(h_hbm.at[src_u.at[b]], rows_v.at[b],
                                  sems.at[b]).wait()
            pltpu.sync_copy(rows_v.at[b], agg_sp.at[dst_u.at[b]], add=True)

            @pl.when(ch + NBUF < NGRP)
            def _():
                unpack(ch + NBUF, b)
                pltpu.async_copy(h_hbm.at[src_u.at[b]], rows_v.at[b],
                                 sems.at[b])

        return 0

    lax.fori_loop(0, NGRP // NBUF, group, 0)
    plsc.subcore_barrier()
    pltpu.sync_copy(agg_sp.at[pl.ds(r0, RPT)], out_hbm.at[c, pl.ds(r0, RPT)])

    @pl.when(s == NS - 1)
    def _():
        pltpu.sync_copy(agg_sp.at[pl.ds(NS * RPT, RREM)],
                        out_hbm.at[c, pl.ds(NS * RPT, RREM)])


def _dense1_body(x_ref, w_ref, hist_ref, h_ref, dinv_ref):
    deg = 1.0 + hist_ref[:, 0:1] + hist_ref[:, 1:2]
    dinv = lax.rsqrt(deg)
    h = jnp.dot(x_ref[...], w_ref[...], preferred_element_type=jnp.float32)
    h_ref[...] = h * dinv
    dinv_ref[...] = dinv


def _mid_body(agg_ref, h_ref, dinv_ref, b_ref, g_ref, be_ref, w_ref, out_ref):
    dinv = dinv_ref[...]
    t = dinv * (agg_ref[0] + agg_ref[1] - h_ref[...]) + b_ref[...]
    t = g_ref[...] * (t * BN_SCALE) + be_ref[...]
    t = jnp.maximum(t, 0.0)
    out_ref[...] = dinv * jnp.dot(t, w_ref[...],
                                  preferred_element_type=jnp.float32)


def _fin_body(agg_ref, h_ref, dinv_ref, b_ref, out_ref):
    out_ref[...] = (dinv_ref[...] * (agg_ref[0] + agg_ref[1] - h_ref[...])
                    + b_ref[...])


_dense1 = pl.pallas_call(
    _dense1_body,
    out_shape=(jax.ShapeDtypeStruct((N, D), jnp.float32),
               jax.ShapeDtypeStruct((N, 1), jnp.float32)),
)

_mid = pl.pallas_call(
    _mid_body,
    out_shape=jax.ShapeDtypeStruct((N, D), jnp.float32),
)

_fin = pl.pallas_call(
    _fin_body,
    out_shape=jax.ShapeDtypeStruct((N, D), jnp.float32),
)


def kernel(x, edge_index, W1, b1, g1, be1, W2, b2, g2, be2, W3, b3):
    src = edge_index[0]
    dst = edge_index[1]
    npad = E_PAD - E
    idxp = jnp.concatenate(
        [src * 65536 + dst,
         jnp.full((npad,), N, jnp.int32)]).reshape(NW, NGRP, ECHUNK)

    hist = _deg_sc(dst)
    histT = hist.T
    h1p, dinv = _dense1(x, W1, histT)
    agg1 = _agg_sc(h1p, idxp)
    h2p = _mid(agg1, h1p, dinv, b1.reshape(1, D), g1.reshape(1, D),
               be1.reshape(1, D), W2)
    agg2 = _agg_sc(h2p, idxp)
    h3p = _mid(agg2, h2p, dinv, b2.reshape(1, D), g2.reshape(1, D),
               be2.reshape(1, D), W3)
    agg3 = _agg_sc(h3p, idxp)
    return _fin(agg3, h3p, dinv, b3.reshape(1, D))


# speedup vs baseline: 1.0943x; 1.0943x over previous
import functools

import jax
import jax.numpy as jnp
from jax import lax
from jax.experimental import pallas as pl
from jax.experimental.pallas import tpu as pltpu
from jax.experimental.pallas import tpu_sc as plsc

N = 10000
E = 320000
D = 128

NC = 2
NS = 16
NW = NC * NS
CHUNK = 80
EPT = E // NW
NCHUNK = EPT // CHUNK
RPT = 624
RREM = N - NS * RPT
BN_SCALE = float(1.0 / (1.0 + 1e-5) ** 0.5)

ECHUNK = 128
NGRP = 80
NBUF = 2
E_PAD = NW * NGRP * ECHUNK
N_SP = N + 8

_sc_mesh = plsc.VectorSubcoreMesh(core_axis_name="c", subcore_axis_name="s")


@functools.partial(
    pl.kernel,
    out_type=jax.ShapeDtypeStruct((NC, N), jnp.float32),
    mesh=_sc_mesh,
    scratch_types=[
        pltpu.VMEM_SHARED((N,), jnp.float32),
        pltpu.VMEM((CHUNK,), jnp.int32),
        pltpu.VMEM((CHUNK,), jnp.float32),
        pltpu.VMEM((N,), jnp.float32),
    ],
)
def _deg_sc(dst_hbm, hist_hbm, hist_sp, dst_v, ones_v, stage_v):
    c = lax.axis_index("c")
    s = lax.axis_index("s")
    wid = c * NS + s

    def fill_ones(i, _):
        ones_v[pl.ds(i * 16, 16)] = jnp.ones((16,), jnp.float32)
        return 0

    lax.fori_loop(0, CHUNK // 16, fill_ones, 0)

    @pl.when(s == 0)
    def _():
        def zero(i, _):
            stage_v[pl.ds(i * 16, 16)] = jnp.zeros((16,), jnp.float32)
            return 0

        lax.fori_loop(0, N // 16, zero, 0)
        pltpu.sync_copy(stage_v, hist_sp)

    plsc.subcore_barrier()

    def body(i, _):
        base = wid * EPT + i * CHUNK
        pltpu.sync_copy(dst_hbm.at[pl.ds(base, CHUNK)], dst_v)
        pltpu.sync_copy(ones_v, hist_sp.at[dst_v], add=True)
        return 0

    lax.fori_loop(0, NCHUNK, body, 0)
    plsc.subcore_barrier()

    @pl.when(s == 0)
    def _():
        pltpu.sync_copy(hist_sp, stage_v)
        pltpu.sync_copy(stage_v, hist_hbm.at[c])


@functools.partial(
    pl.kernel,
    out_type=jax.ShapeDtypeStruct((NC, N, D), jnp.float32),
    mesh=_sc_mesh,
    scratch_types=[
        pltpu.VMEM_SHARED((N_SP, D), jnp.float32),
        pltpu.VMEM((NGRP, ECHUNK), jnp.int32),
        pltpu.VMEM((NBUF, ECHUNK), jnp.int32),
        pltpu.VMEM((NBUF, ECHUNK), jnp.int32),
        pltpu.VMEM((NBUF, ECHUNK, D), jnp.float32),
        pltpu.SemaphoreType.DMA((NBUF,)),
    ],
)
def _agg_sc(h_hbm, idx_hbm, out_hbm, agg_sp, idx_v, src_u, dst_u, rows_v,
            sems):
    c = lax.axis_index("c")
    s = lax.axis_index("s")
    wid = c * NS + s

    pltpu.sync_copy(idx_hbm.at[wid], idx_v)
    r0 = s * RPT
    pltpu.sync_copy(h_hbm.at[pl.ds(r0, RPT)], agg_sp.at[pl.ds(r0, RPT)])

    @pl.when(s == NS - 1)
    def _():
        pltpu.sync_copy(h_hbm.at[pl.ds(NS * RPT, RREM)],
                        agg_sp.at[pl.ds(NS * RPT, RREM)])

    def unpack(ch, b):
        for k in range(ECHUNK // 16):
            w = idx_v[ch, pl.ds(k * 16, 16)]
            src_u[b, pl.ds(k * 16, 16)] = lax.shift_right_logical(w, 16)
            dst_u[b, pl.ds(k * 16, 16)] = lax.bitwise_and(w, 0xFFFF)

    plsc.subcore_barrier()

    for b in range(NBUF):
        unpack(b, b)
        pltpu.async_copy(h_hbm.at[src_u.at[b]], rows_v.at[b], sems.at[b])

    def group(g, _):
        for b in range(NBUF):
            ch = g * NBUF + b
            pltpu.make_async_copy(h_hbm.at[src_u.at[b]], rows_v.at[b],
                                  sems.at[b]).wait()
            pltpu.sync_copy(rows_v.at[b], agg_sp.at[dst_u.at[b]], add=True)

            @pl.when(ch + NBUF < NGRP)
            def _():
                unpack(ch + NBUF, b)
                pltpu.async_copy(h_hbm.at[src_u.at[b]], rows_v.at[b],
                                 sems.at[b])

        return 0

    lax.fori_loop(0, NGRP // NBUF, group, 0)
    plsc.subcore_barrier()
    pltpu.sync_copy(agg_sp.at[pl.ds(r0, RPT)], out_hbm.at[c, pl.ds(r0, RPT)])

    @pl.when(s == NS - 1)
    def _():
        pltpu.sync_copy(agg_sp.at[pl.ds(NS * RPT, RREM)],
                        out_hbm.at[c, pl.ds(NS * RPT, RREM)])


def _dense1_body(x_ref, w_ref, hist_ref, h_ref, dinv_ref):
    deg = 1.0 + hist_ref[:, 0:1] + hist_ref[:, 1:2]
    dinv = lax.rsqrt(deg)
    h = jnp.dot(x_ref[...], w_ref[...], preferred_element_type=jnp.float32)
    h_ref[...] = h * dinv
    dinv_ref[...] = dinv


def _mid_body(agg_ref, h_ref, dinv_ref, b_ref, g_ref, be_ref, w_ref, out_ref):
    dinv = dinv_ref[...]
    t = dinv * (agg_ref[0] + agg_ref[1] - h_ref[...]) + b_ref[...]
    t = g_ref[...] * (t * BN_SCALE) + be_ref[...]
    t = jnp.maximum(t, 0.0)
    out_ref[...] = dinv * jnp.dot(t, w_ref[...],
                                  preferred_element_type=jnp.float32)


def _fin_body(agg_ref, h_ref, dinv_ref, b_ref, out_ref):
    out_ref[...] = (dinv_ref[...] * (agg_ref[0] + agg_ref[1] - h_ref[...])
                    + b_ref[...])


_dense1 = pl.pallas_call(
    _dense1_body,
    out_shape=(jax.ShapeDtypeStruct((N, D), jnp.float32),
               jax.ShapeDtypeStruct((N, 1), jnp.float32)),
)

_mid = pl.pallas_call(
    _mid_body,
    out_shape=jax.ShapeDtypeStruct((N, D), jnp.float32),
)

_fin = pl.pallas_call(
    _fin_body,
    out_shape=jax.ShapeDtypeStruct((N, D), jnp.float32),
)


def kernel(x, edge_index, W1, b1, g1, be1, W2, b2, g2, be2, W3, b3):
    src = edge_index[0]
    dst = edge_index[1]
    ppt = (E_PAD - E) // NW
    packed = (src * 65536 + dst).reshape(NW, E // NW)
    pad = jnp.broadcast_to(N + jnp.arange(ppt, dtype=jnp.int32) % 8,
                           (NW, ppt))
    idxp = jnp.concatenate([packed, pad], axis=1).reshape(NW, NGRP, ECHUNK)

    hist = _deg_sc(dst)
    histT = hist.T
    h1p, dinv = _dense1(x, W1, histT)
    agg1 = _agg_sc(h1p, idxp)
    h2p = _mid(agg1, h1p, dinv, b1.reshape(1, D), g1.reshape(1, D),
               be1.reshape(1, D), W2)
    agg2 = _agg_sc(h2p, idxp)
    h3p = _mid(agg2, h2p, dinv, b2.reshape(1, D), g2.reshape(1, D),
               be2.reshape(1, D), W3)
    agg3 = _agg_sc(h3p, idxp)
    return _fin(agg3, h3p, dinv, b3.reshape(1, D))


# speedup vs baseline: 3.2645x; 2.9833x over previous
import functools

import jax
import jax.numpy as jnp
from jax import lax
from jax.experimental import pallas as pl
from jax.experimental.pallas import tpu as pltpu
from jax.experimental.pallas import tpu_sc as plsc

N = 10000
E = 320000
D = 128

NC = 2
NS = 16
NW = NC * NS
CHUNK = 80
EPT = E // NW
NCHUNK = EPT // CHUNK
RPT = 624
RREM = N - NS * RPT
BN_SCALE = float(1.0 / (1.0 + 1e-5) ** 0.5)

ECHUNK = 80
NGRP = 125
NBUF = 3
N_SP = N + 8

_sc_mesh = plsc.VectorSubcoreMesh(core_axis_name="c", subcore_axis_name="s")


@functools.partial(
    pl.kernel,
    out_type=jax.ShapeDtypeStruct((NC, N), jnp.float32),
    mesh=_sc_mesh,
    scratch_types=[
        pltpu.VMEM_SHARED((N,), jnp.float32),
        pltpu.VMEM((CHUNK,), jnp.int32),
        pltpu.VMEM((CHUNK,), jnp.float32),
        pltpu.VMEM((N,), jnp.float32),
    ],
)
def _deg_sc(dst_hbm, hist_hbm, hist_sp, dst_v, ones_v, stage_v):
    c = lax.axis_index("c")
    s = lax.axis_index("s")
    wid = c * NS + s

    def fill_ones(i, _):
        ones_v[pl.ds(i * 16, 16)] = jnp.ones((16,), jnp.float32)
        return 0

    lax.fori_loop(0, CHUNK // 16, fill_ones, 0)

    @pl.when(s == 0)
    def _():
        def zero(i, _):
            stage_v[pl.ds(i * 16, 16)] = jnp.zeros((16,), jnp.float32)
            return 0

        lax.fori_loop(0, N // 16, zero, 0)
        pltpu.sync_copy(stage_v, hist_sp)

    plsc.subcore_barrier()

    def body(i, _):
        base = wid * EPT + i * CHUNK
        pltpu.sync_copy(dst_hbm.at[pl.ds(base, CHUNK)], dst_v)
        pltpu.sync_copy(ones_v, hist_sp.at[dst_v], add=True)
        return 0

    lax.fori_loop(0, NCHUNK, body, 0)
    plsc.subcore_barrier()

    @pl.when(s == 0)
    def _():
        pltpu.sync_copy(hist_sp, stage_v)
        pltpu.sync_copy(stage_v, hist_hbm.at[c])


@functools.partial(
    pl.kernel,
    out_type=jax.ShapeDtypeStruct((NC, N, D), jnp.float32),
    mesh=_sc_mesh,
    scratch_types=[
        pltpu.VMEM_SHARED((N_SP, D), jnp.float32),
        pltpu.VMEM((NGRP, ECHUNK), jnp.int32),
        pltpu.VMEM((NBUF, ECHUNK), jnp.int32),
        pltpu.VMEM((NBUF, ECHUNK), jnp.int32),
        pltpu.VMEM((NBUF, ECHUNK, D), jnp.float32),
        pltpu.SemaphoreType.DMA((NBUF,)),
    ],
)
def _agg_sc(h_hbm, idx_hbm, out_hbm, agg_sp, idx_v, src_u, dst_u, rows_v,
            sems):
    c = lax.axis_index("c")
    s = lax.axis_index("s")
    wid = c * NS + s

    pltpu.sync_copy(idx_hbm.at[wid], idx_v)
    r0 = s * RPT
    pltpu.sync_copy(h_hbm.at[pl.ds(r0, RPT)], agg_sp.at[pl.ds(r0, RPT)])

    @pl.when(s == NS - 1)
    def _():
        pltpu.sync_copy(h_hbm.at[pl.ds(NS * RPT, RREM)],
                        agg_sp.at[pl.ds(NS * RPT, RREM)])

    def unpack(ch, b):
        for k in range(ECHUNK // 16):
            w = idx_v[ch, pl.ds(k * 16, 16)]
            src_u[b, pl.ds(k * 16, 16)] = lax.shift_right_logical(w, 16)
            dst_u[b, pl.ds(k * 16, 16)] = lax.bitwise_and(w, 0xFFFF)

    plsc.subcore_barrier()

    for b in range(NBUF - 1):
        unpack(b, b)
        pltpu.async_copy(h_hbm.at[src_u.at[b]], rows_v.at[b], sems.at[b])

    def group(g, _):
        for b in range(NBUF):
            ch = g * NBUF + b
            b2 = (b + NBUF - 1) % NBUF

            @pl.when(ch < NGRP)
            def _():
                pltpu.make_async_copy(h_hbm.at[src_u.at[b]], rows_v.at[b],
                                      sems.at[b]).wait()

            @pl.when(ch + NBUF - 1 < NGRP)
            def _():
                unpack(ch + NBUF - 1, b2)
                pltpu.async_copy(h_hbm.at[src_u.at[b2]], rows_v.at[b2],
                                 sems.at[b2])

            @pl.when(ch < NGRP)
            def _():
                pltpu.sync_copy(rows_v.at[b], agg_sp.at[dst_u.at[b]],
                                add=True)

        return 0

    lax.fori_loop(0, pl.cdiv(NGRP, NBUF), group, 0)
    plsc.subcore_barrier()
    pltpu.sync_copy(agg_sp.at[pl.ds(r0, RPT)], out_hbm.at[c, pl.ds(r0, RPT)])

    @pl.when(s == NS - 1)
    def _():
        pltpu.sync_copy(agg_sp.at[pl.ds(NS * RPT, RREM)],
                        out_hbm.at[c, pl.ds(NS * RPT, RREM)])


def _dense1_body(x_ref, w_ref, hist_ref, h_ref, dinv_ref):
    deg = 1.0 + hist_ref[:, 0:1] + hist_ref[:, 1:2]
    dinv = lax.rsqrt(deg)
    h = jnp.dot(x_ref[...], w_ref[...], preferred_element_type=jnp.float32)
    h_ref[...] = h * dinv
    dinv_ref[...] = dinv


def _mid_body(agg_ref, h_ref, dinv_ref, b_ref, g_ref, be_ref, w_ref, out_ref):
    dinv = dinv_ref[...]
    t = dinv * (agg_ref[0] + agg_ref[1] - h_ref[...]) + b_ref[...]
    t = g_ref[...] * (t * BN_SCALE) + be_ref[...]
    t = jnp.maximum(t, 0.0)
    out_ref[...] = dinv * jnp.dot(t, w_ref[...],
                                  preferred_element_type=jnp.float32)


def _fin_body(agg_ref, h_ref, dinv_ref, b_ref, out_ref):
    out_ref[...] = (dinv_ref[...] * (agg_ref[0] + agg_ref[1] - h_ref[...])
                    + b_ref[...])


_dense1 = pl.pallas_call(
    _dense1_body,
    out_shape=(jax.ShapeDtypeStruct((N, D), jnp.float32),
               jax.ShapeDtypeStruct((N, 1), jnp.float32)),
)

_mid = pl.pallas_call(
    _mid_body,
    out_shape=jax.ShapeDtypeStruct((N, D), jnp.float32),
)

_fin = pl.pallas_call(
    _fin_body,
    out_shape=jax.ShapeDtypeStruct((N, D), jnp.float32),
)


def kernel(x, edge_index, W1, b1, g1, be1, W2, b2, g2, be2, W3, b3):
    src = edge_index[0]
    dst = edge_index[1]
    idxp = (src * 65536 + dst).reshape(NW, NGRP, ECHUNK)

    hist = _deg_sc(dst)
    histT = hist.T
    h1p, dinv = _dense1(x, W1, histT)
    agg1 = _agg_sc(h1p, idxp)
    h2p = _mid(agg1, h1p, dinv, b1.reshape(1, D), g1.reshape(1, D),
               be1.reshape(1, D), W2)
    agg2 = _agg_sc(h2p, idxp)
    h3p = _mid(agg2, h2p, dinv, b2.reshape(1, D), g2.reshape(1, D),
               be2.reshape(1, D), W3)
    agg3 = _agg_sc(h3p, idxp)
    return _fin(agg3, h3p, dinv, b3.reshape(1, D))


# speedup vs baseline: 3.6856x; 1.1290x over previous
import functools

import jax
import jax.numpy as jnp
from jax import lax
from jax.experimental import pallas as pl
from jax.experimental.pallas import tpu as pltpu
from jax.experimental.pallas import tpu_sc as plsc

N = 10000
E = 320000
D = 128

NC = 2
NS = 16
NW = NC * NS
CHUNK = 80
EPT = E // NW
NCHUNK = EPT // CHUNK
RPT = 624
RREM = N - NS * RPT
BN_SCALE = float(1.0 / (1.0 + 1e-5) ** 0.5)

ECHUNK = 80
NGRP = 125
NBUF = 3
N_SP = N + 8

_sc_mesh = plsc.VectorSubcoreMesh(core_axis_name="c", subcore_axis_name="s")


@functools.partial(
    pl.kernel,
    out_type=jax.ShapeDtypeStruct((NC * N,), jnp.float32),
    mesh=_sc_mesh,
    scratch_types=[
        pltpu.VMEM_SHARED((N,), jnp.float32),
        pltpu.VMEM((NGRP, ECHUNK), jnp.int32),
        pltpu.VMEM((ECHUNK,), jnp.int32),
        pltpu.VMEM((ECHUNK,), jnp.float32),
        pltpu.VMEM((RPT,), jnp.float32),
    ],
)
def _deg_sc(idx_hbm, hist_hbm, hist_sp, idx_v, dst_u, ones_v, zero_v):
    c = lax.axis_index("c")
    s = lax.axis_index("s")
    wid = c * NS + s

    pltpu.sync_copy(idx_hbm.at[wid], idx_v)

    for i in range(ECHUNK // 16):
        ones_v[pl.ds(i * 16, 16)] = jnp.ones((16,), jnp.float32)

    def zero(i, _):
        zero_v[pl.ds(i * 16, 16)] = jnp.zeros((16,), jnp.float32)
        return 0

    lax.fori_loop(0, RPT // 16, zero, 0)
    r0 = s * RPT
    pltpu.sync_copy(zero_v, hist_sp.at[pl.ds(r0, RPT)])

    @pl.when(s == NS - 1)
    def _():
        pltpu.sync_copy(zero_v.at[pl.ds(0, RREM)],
                        hist_sp.at[pl.ds(NS * RPT, RREM)])

    plsc.subcore_barrier()

    def body(ch, _):
        for k in range(ECHUNK // 16):
            w = idx_v[ch, pl.ds(k * 16, 16)]
            dst_u[pl.ds(k * 16, 16)] = lax.bitwise_and(w, 0xFFFF)
        pltpu.sync_copy(ones_v, hist_sp.at[dst_u], add=True)
        return 0

    lax.fori_loop(0, NGRP, body, 0)
    plsc.subcore_barrier()
    pltpu.sync_copy(hist_sp.at[pl.ds(r0, RPT)], zero_v)
    pltpu.sync_copy(zero_v, hist_hbm.at[pl.ds(c * N + r0, RPT)])

    @pl.when(s == NS - 1)
    def _():
        pltpu.sync_copy(hist_sp.at[pl.ds(NS * RPT, RREM)],
                        zero_v.at[pl.ds(0, RREM)])
        pltpu.sync_copy(zero_v.at[pl.ds(0, RREM)],
                        hist_hbm.at[pl.ds(c * N + NS * RPT, RREM)])


@functools.partial(
    pl.kernel,
    out_type=jax.ShapeDtypeStruct((NC, N, D), jnp.float32),
    mesh=_sc_mesh,
    scratch_types=[
        pltpu.VMEM_SHARED((N_SP, D), jnp.float32),
        pltpu.VMEM((NGRP, ECHUNK), jnp.int32),
        pltpu.VMEM((NBUF, ECHUNK), jnp.int32),
        pltpu.VMEM((NBUF, ECHUNK), jnp.int32),
        pltpu.VMEM((NBUF, ECHUNK, D), jnp.float32),
        pltpu.SemaphoreType.DMA((NBUF,)),
    ],
)
def _agg_sc(h_hbm, idx_hbm, out_hbm, agg_sp, idx_v, src_u, dst_u, rows_v,
            sems):
    c = lax.axis_index("c")
    s = lax.axis_index("s")
    wid = c * NS + s

    pltpu.sync_copy(idx_hbm.at[wid], idx_v)
    r0 = s * RPT
    pltpu.sync_copy(h_hbm.at[pl.ds(r0, RPT)], agg_sp.at[pl.ds(r0, RPT)])

    @pl.when(s == NS - 1)
    def _():
        pltpu.sync_copy(h_hbm.at[pl.ds(NS * RPT, RREM)],
                        agg_sp.at[pl.ds(NS * RPT, RREM)])

    def unpack(ch, b):
        for k in range(ECHUNK // 16):
            w = idx_v[ch, pl.ds(k * 16, 16)]
            src_u[b, pl.ds(k * 16, 16)] = lax.shift_right_logical(w, 16)
            dst_u[b, pl.ds(k * 16, 16)] = lax.bitwise_and(w, 0xFFFF)

    plsc.subcore_barrier()

    for b in range(NBUF - 1):
        unpack(b, b)
        pltpu.async_copy(h_hbm.at[src_u.at[b]], rows_v.at[b], sems.at[b])

    def group(g, _):
        for b in range(NBUF):
            ch = g * NBUF + b
            b2 = (b + NBUF - 1) % NBUF

            @pl.when(ch < NGRP)
            def _():
                pltpu.make_async_copy(h_hbm.at[src_u.at[b]], rows_v.at[b],
                                      sems.at[b]).wait()

            @pl.when(ch + NBUF - 1 < NGRP)
            def _():
                unpack(ch + NBUF - 1, b2)
                pltpu.async_copy(h_hbm.at[src_u.at[b2]], rows_v.at[b2],
                                 sems.at[b2])

            @pl.when(ch < NGRP)
            def _():
                pltpu.sync_copy(rows_v.at[b], agg_sp.at[dst_u.at[b]],
                                add=True)

        return 0

    lax.fori_loop(0, pl.cdiv(NGRP, NBUF), group, 0)
    plsc.subcore_barrier()
    pltpu.sync_copy(agg_sp.at[pl.ds(r0, RPT)], out_hbm.at[c, pl.ds(r0, RPT)])

    @pl.when(s == NS - 1)
    def _():
        pltpu.sync_copy(agg_sp.at[pl.ds(NS * RPT, RREM)],
                        out_hbm.at[c, pl.ds(NS * RPT, RREM)])


def _dense1_body(x_ref, w_ref, hist_ref, h_ref, dinv_ref):
    deg = 1.0 + hist_ref[:, 0:1] + hist_ref[:, 1:2]
    dinv = lax.rsqrt(deg)
    h = jnp.dot(x_ref[...], w_ref[...], preferred_element_type=jnp.float32)
    h_ref[...] = h * dinv
    dinv_ref[...] = dinv


def _mid_body(agg_ref, h_ref, dinv_ref, b_ref, g_ref, be_ref, w_ref, out_ref):
    dinv = dinv_ref[...]
    t = dinv * (agg_ref[0] + agg_ref[1] - h_ref[...]) + b_ref[...]
    t = g_ref[...] * (t * BN_SCALE) + be_ref[...]
    t = jnp.maximum(t, 0.0)
    out_ref[...] = dinv * jnp.dot(t, w_ref[...],
                                  preferred_element_type=jnp.float32)


def _fin_body(agg_ref, h_ref, dinv_ref, b_ref, out_ref):
    out_ref[...] = (dinv_ref[...] * (agg_ref[0] + agg_ref[1] - h_ref[...])
                    + b_ref[...])


_dense1 = pl.pallas_call(
    _dense1_body,
    out_shape=(jax.ShapeDtypeStruct((N, D), jnp.float32),
               jax.ShapeDtypeStruct((N, 1), jnp.float32)),
)

_mid = pl.pallas_call(
    _mid_body,
    out_shape=jax.ShapeDtypeStruct((N, D), jnp.float32),
)

_fin = pl.pallas_call(
    _fin_body,
    out_shape=jax.ShapeDtypeStruct((N, D), jnp.float32),
)


def kernel(x, edge_index, W1, b1, g1, be1, W2, b2, g2, be2, W3, b3):
    src = edge_index[0]
    dst = edge_index[1]
    idxp = (src * 65536 + dst).reshape(NW, NGRP, ECHUNK)

    hist = _deg_sc(idxp)
    histT = hist.reshape(NC, N).T
    h1p, dinv = _dense1(x, W1, histT)
    agg1 = _agg_sc(h1p, idxp)
    h2p = _mid(agg1, h1p, dinv, b1.reshape(1, D), g1.reshape(1, D),
               be1.reshape(1, D), W2)
    agg2 = _agg_sc(h2p, idxp)
    h3p = _mid(agg2, h2p, dinv, b2.reshape(1, D), g2.reshape(1, D),
               be2.reshape(1, D), W3)
    agg3 = _agg_sc(h3p, idxp)
    return _fin(agg3, h3p, dinv, b3.reshape(1, D))


# speedup vs baseline: 3.7768x; 1.0247x over previous
import functools

import jax
import jax.numpy as jnp
from jax import lax
from jax.experimental import pallas as pl
from jax.experimental.pallas import tpu as pltpu
from jax.experimental.pallas import tpu_sc as plsc

N = 10000
E = 320000
D = 128

NC = 2
NS = 16
NW = NC * NS
CHUNK = 80
EPT = E // NW
NCHUNK = EPT // CHUNK
RPT = 624
RREM = N - NS * RPT
BN_SCALE = float(1.0 / (1.0 + 1e-5) ** 0.5)

ECHUNK = 80
NGRP = 125
NBUF = 3
N_SP = N + 8

_sc_mesh = plsc.VectorSubcoreMesh(core_axis_name="c", subcore_axis_name="s")


@functools.partial(
    pl.kernel,
    out_type=(jax.ShapeDtypeStruct((E,), jnp.int32),
              jax.ShapeDtypeStruct((NC * N,), jnp.float32)),
    mesh=_sc_mesh,
    scratch_types=[
        pltpu.VMEM_SHARED((N,), jnp.float32),
        pltpu.VMEM((EPT,), jnp.int32),
        pltpu.VMEM((EPT,), jnp.int32),
        pltpu.VMEM((ECHUNK,), jnp.int32),
        pltpu.VMEM((ECHUNK,), jnp.float32),
        pltpu.VMEM((RPT,), jnp.float32),
    ],
)
def _deg_sc(ei_hbm, idxp_hbm, hist_hbm, hist_sp, src_v, dst_v, dst_u, ones_v,
            zero_v):
    c = lax.axis_index("c")
    s = lax.axis_index("s")
    wid = c * NS + s
    e0 = wid * EPT

    pltpu.sync_copy(ei_hbm.at[pl.ds(e0, EPT)], src_v)
    pltpu.sync_copy(ei_hbm.at[pl.ds(E + e0, EPT)], dst_v)

    for i in range(ECHUNK // 16):
        ones_v[pl.ds(i * 16, 16)] = jnp.ones((16,), jnp.float32)

    def zero(i, _):
        zero_v[pl.ds(i * 16, 16)] = jnp.zeros((16,), jnp.float32)
        return 0

    lax.fori_loop(0, RPT // 16, zero, 0)
    r0 = s * RPT
    pltpu.sync_copy(zero_v, hist_sp.at[pl.ds(r0, RPT)])

    @pl.when(s == NS - 1)
    def _():
        pltpu.sync_copy(zero_v.at[pl.ds(0, RREM)],
                        hist_sp.at[pl.ds(NS * RPT, RREM)])

    def pack(i, _):
        sl = pl.ds(i * 16, 16)
        src_v[sl] = lax.bitwise_or(lax.shift_left(src_v[sl], 16), dst_v[sl])
        return 0

    lax.fori_loop(0, EPT // 16, pack, 0)
    pltpu.sync_copy(src_v, idxp_hbm.at[pl.ds(e0, EPT)])

    plsc.subcore_barrier()

    def body(ch, _):
        for k in range(ECHUNK // 16):
            dst_u[pl.ds(k * 16, 16)] = dst_v[pl.ds(ch * ECHUNK + k * 16, 16)]
        pltpu.sync_copy(ones_v, hist_sp.at[dst_u], add=True)
        return 0

    lax.fori_loop(0, NGRP, body, 0)
    plsc.subcore_barrier()
    pltpu.sync_copy(hist_sp.at[pl.ds(r0, RPT)], zero_v)
    pltpu.sync_copy(zero_v, hist_hbm.at[pl.ds(c * N + r0, RPT)])

    @pl.when(s == NS - 1)
    def _():
        pltpu.sync_copy(hist_sp.at[pl.ds(NS * RPT, RREM)],
                        zero_v.at[pl.ds(0, RREM)])
        pltpu.sync_copy(zero_v.at[pl.ds(0, RREM)],
                        hist_hbm.at[pl.ds(c * N + NS * RPT, RREM)])


@functools.partial(
    pl.kernel,
    out_type=jax.ShapeDtypeStruct((NC, N, D), jnp.float32),
    mesh=_sc_mesh,
    scratch_types=[
        pltpu.VMEM_SHARED((N_SP, D), jnp.float32),
        pltpu.VMEM((EPT,), jnp.int32),
        pltpu.VMEM((NBUF, ECHUNK), jnp.int32),
        pltpu.VMEM((NBUF, ECHUNK), jnp.int32),
        pltpu.VMEM((NBUF, ECHUNK, D), jnp.float32),
        pltpu.SemaphoreType.DMA((NBUF,)),
    ],
)
def _agg_sc(h_hbm, idx_hbm, out_hbm, agg_sp, idx_v, src_u, dst_u, rows_v,
            sems):
    c = lax.axis_index("c")
    s = lax.axis_index("s")
    wid = c * NS + s

    pltpu.sync_copy(idx_hbm.at[pl.ds(wid * EPT, EPT)], idx_v)
    r0 = s * RPT
    pltpu.sync_copy(h_hbm.at[pl.ds(r0, RPT)], agg_sp.at[pl.ds(r0, RPT)])

    @pl.when(s == NS - 1)
    def _():
        pltpu.sync_copy(h_hbm.at[pl.ds(NS * RPT, RREM)],
                        agg_sp.at[pl.ds(NS * RPT, RREM)])

    def unpack(ch, b):
        for k in range(ECHUNK // 16):
            w = idx_v[pl.ds(ch * ECHUNK + k * 16, 16)]
            src_u[b, pl.ds(k * 16, 16)] = lax.shift_right_logical(w, 16)
            dst_u[b, pl.ds(k * 16, 16)] = lax.bitwise_and(w, 0xFFFF)

    plsc.subcore_barrier()

    for b in range(NBUF - 1):
        unpack(b, b)
        pltpu.async_copy(h_hbm.at[src_u.at[b]], rows_v.at[b], sems.at[b])

    def group(g, _):
        for b in range(NBUF):
            ch = g * NBUF + b
            b2 = (b + NBUF - 1) % NBUF

            @pl.when(ch < NGRP)
            def _():
                pltpu.make_async_copy(h_hbm.at[src_u.at[b]], rows_v.at[b],
                                      sems.at[b]).wait()

            @pl.when(ch + NBUF - 1 < NGRP)
            def _():
                unpack(ch + NBUF - 1, b2)
                pltpu.async_copy(h_hbm.at[src_u.at[b2]], rows_v.at[b2],
                                 sems.at[b2])

            @pl.when(ch < NGRP)
            def _():
                pltpu.sync_copy(rows_v.at[b], agg_sp.at[dst_u.at[b]],
                                add=True)

        return 0

    lax.fori_loop(0, pl.cdiv(NGRP, NBUF), group, 0)
    plsc.subcore_barrier()
    pltpu.sync_copy(agg_sp.at[pl.ds(r0, RPT)], out_hbm.at[c, pl.ds(r0, RPT)])

    @pl.when(s == NS - 1)
    def _():
        pltpu.sync_copy(agg_sp.at[pl.ds(NS * RPT, RREM)],
                        out_hbm.at[c, pl.ds(NS * RPT, RREM)])


def _dense1_body(x_ref, w_ref, hist_ref, h_ref, dinv_ref):
    deg = 1.0 + hist_ref[:, 0:1] + hist_ref[:, 1:2]
    dinv = lax.rsqrt(deg)
    h = jnp.dot(x_ref[...], w_ref[...], preferred_element_type=jnp.float32)
    h_ref[...] = h * dinv
    dinv_ref[...] = dinv


def _mid_body(agg_ref, h_ref, dinv_ref, b_ref, g_ref, be_ref, w_ref, out_ref):
    dinv = dinv_ref[...]
    t = dinv * (agg_ref[0] + agg_ref[1] - h_ref[...]) + b_ref[...]
    t = g_ref[...] * (t * BN_SCALE) + be_ref[...]
    t = jnp.maximum(t, 0.0)
    out_ref[...] = dinv * jnp.dot(t, w_ref[...],
                                  preferred_element_type=jnp.float32)


def _fin_body(agg_ref, h_ref, dinv_ref, b_ref, out_ref):
    out_ref[...] = (dinv_ref[...] * (agg_ref[0] + agg_ref[1] - h_ref[...])
                    + b_ref[...])


RB = 2000
_GRID = N // RB

_dense1 = pl.pallas_call(
    _dense1_body,
    grid=(_GRID,),
    in_specs=[
        pl.BlockSpec((RB, D), lambda i: (i, 0)),
        pl.BlockSpec((D, D), lambda i: (0, 0)),
        pl.BlockSpec((RB, 2), lambda i: (i, 0)),
    ],
    out_specs=(pl.BlockSpec((RB, D), lambda i: (i, 0)),
               pl.BlockSpec((RB, 1), lambda i: (i, 0))),
    out_shape=(jax.ShapeDtypeStruct((N, D), jnp.float32),
               jax.ShapeDtypeStruct((N, 1), jnp.float32)),
)

_mid = pl.pallas_call(
    _mid_body,
    grid=(_GRID,),
    in_specs=[
        pl.BlockSpec((NC, RB, D), lambda i: (0, i, 0)),
        pl.BlockSpec((RB, D), lambda i: (i, 0)),
        pl.BlockSpec((RB, 1), lambda i: (i, 0)),
        pl.BlockSpec((1, D), lambda i: (0, 0)),
        pl.BlockSpec((1, D), lambda i: (0, 0)),
        pl.BlockSpec((1, D), lambda i: (0, 0)),
        pl.BlockSpec((D, D), lambda i: (0, 0)),
    ],
    out_specs=pl.BlockSpec((RB, D), lambda i: (i, 0)),
    out_shape=jax.ShapeDtypeStruct((N, D), jnp.float32),
)

_fin = pl.pallas_call(
    _fin_body,
    grid=(_GRID,),
    in_specs=[
        pl.BlockSpec((NC, RB, D), lambda i: (0, i, 0)),
        pl.BlockSpec((RB, D), lambda i: (i, 0)),
        pl.BlockSpec((RB, 1), lambda i: (i, 0)),
        pl.BlockSpec((1, D), lambda i: (0, 0)),
    ],
    out_specs=pl.BlockSpec((RB, D), lambda i: (i, 0)),
    out_shape=jax.ShapeDtypeStruct((N, D), jnp.float32),
)


def kernel(x, edge_index, W1, b1, g1, be1, W2, b2, g2, be2, W3, b3):
    ei_flat = edge_index.reshape(2 * E)

    idxp, hist = _deg_sc(ei_flat)
    histT = hist.reshape(NC, N).T
    h1p, dinv = _dense1(x, W1, histT)
    agg1 = _agg_sc(h1p, idxp)
    h2p = _mid(agg1, h1p, dinv, b1.reshape(1, D), g1.reshape(1, D),
               be1.reshape(1, D), W2)
    agg2 = _agg_sc(h2p, idxp)
    h3p = _mid(agg2, h2p, dinv, b2.reshape(1, D), g2.reshape(1, D),
               be2.reshape(1, D), W3)
    agg3 = _agg_sc(h3p, idxp)
    return _fin(agg3, h3p, dinv, b3.reshape(1, D))


# speedup vs baseline: 3.8158x; 1.0103x over previous
import functools

import jax
import jax.numpy as jnp
from jax import lax
from jax.experimental import pallas as pl
from jax.experimental.pallas import tpu as pltpu
from jax.experimental.pallas import tpu_sc as plsc

N = 10000
E = 320000
D = 128

NC = 2
NS = 16
NW = NC * NS
CHUNK = 80
EPT = E // NW
NCHUNK = EPT // CHUNK
RPT = 624
RREM = N - NS * RPT
BN_SCALE = float(1.0 / (1.0 + 1e-5) ** 0.5)

ECHUNK = 80
NGRP = 125
NBUF = 3
N_SP = N + 8

_sc_mesh = plsc.VectorSubcoreMesh(core_axis_name="c", subcore_axis_name="s")


@functools.partial(
    pl.kernel,
    out_type=(jax.ShapeDtypeStruct((E,), jnp.int32),
              jax.ShapeDtypeStruct((NC * N,), jnp.float32)),
    mesh=_sc_mesh,
    scratch_types=[
        pltpu.VMEM_SHARED((N,), jnp.float32),
        pltpu.VMEM((EPT,), jnp.int32),
        pltpu.VMEM((EPT,), jnp.int32),
        pltpu.VMEM((ECHUNK,), jnp.int32),
        pltpu.VMEM((ECHUNK,), jnp.float32),
        pltpu.VMEM((RPT,), jnp.float32),
    ],
)
def _deg_sc(ei_hbm, idxp_hbm, hist_hbm, hist_sp, src_v, dst_v, dst_u, ones_v,
            zero_v):
    c = lax.axis_index("c")
    s = lax.axis_index("s")
    wid = c * NS + s
    e0 = wid * EPT

    pltpu.sync_copy(ei_hbm.at[pl.ds(e0, EPT)], src_v)
    pltpu.sync_copy(ei_hbm.at[pl.ds(E + e0, EPT)], dst_v)

    for i in range(ECHUNK // 16):
        ones_v[pl.ds(i * 16, 16)] = jnp.ones((16,), jnp.float32)

    def zero(i, _):
        zero_v[pl.ds(i * 16, 16)] = jnp.zeros((16,), jnp.float32)
        return 0

    lax.fori_loop(0, RPT // 16, zero, 0)
    r0 = s * RPT
    pltpu.sync_copy(zero_v, hist_sp.at[pl.ds(r0, RPT)])

    @pl.when(s == NS - 1)
    def _():
        pltpu.sync_copy(zero_v.at[pl.ds(0, RREM)],
                        hist_sp.at[pl.ds(NS * RPT, RREM)])

    def pack(i, _):
        sl = pl.ds(i * 16, 16)
        src_v[sl] = lax.bitwise_or(lax.shift_left(src_v[sl], 16), dst_v[sl])
        return 0

    lax.fori_loop(0, EPT // 16, pack, 0)
    pltpu.sync_copy(src_v, idxp_hbm.at[pl.ds(e0, EPT)])

    plsc.subcore_barrier()

    def body(ch, _):
        for k in range(ECHUNK // 16):
            dst_u[pl.ds(k * 16, 16)] = dst_v[pl.ds(ch * ECHUNK + k * 16, 16)]
        pltpu.sync_copy(ones_v, hist_sp.at[dst_u], add=True)
        return 0

    lax.fori_loop(0, NGRP, body, 0)
    plsc.subcore_barrier()
    pltpu.sync_copy(hist_sp.at[pl.ds(r0, RPT)], zero_v)
    pltpu.sync_copy(zero_v, hist_hbm.at[pl.ds(c * N + r0, RPT)])

    @pl.when(s == NS - 1)
    def _():
        pltpu.sync_copy(hist_sp.at[pl.ds(NS * RPT, RREM)],
                        zero_v.at[pl.ds(0, RREM)])
        pltpu.sync_copy(zero_v.at[pl.ds(0, RREM)],
                        hist_hbm.at[pl.ds(c * N + NS * RPT, RREM)])


@functools.partial(
    pl.kernel,
    out_type=jax.ShapeDtypeStruct((NC, N, D), jnp.float32),
    mesh=_sc_mesh,
    scratch_types=[
        pltpu.VMEM_SHARED((N_SP, D), jnp.float32),
        pltpu.VMEM((EPT,), jnp.int32),
        pltpu.VMEM((NBUF, ECHUNK), jnp.int32),
        pltpu.VMEM((NBUF, ECHUNK), jnp.int32),
        pltpu.VMEM((NBUF, ECHUNK, D), jnp.float32),
        pltpu.SemaphoreType.DMA((NBUF,)),
    ],
)
def _agg_sc(h_hbm, idx_hbm, out_hbm, agg_sp, idx_v, src_u, dst_u, rows_v,
            sems):
    c = lax.axis_index("c")
    s = lax.axis_index("s")
    wid = c * NS + s

    pltpu.sync_copy(idx_hbm.at[pl.ds(wid * EPT, EPT)], idx_v)
    r0 = s * RPT

    def unpack(ch, b):
        for k in range(ECHUNK // 16):
            w = idx_v[pl.ds(ch * ECHUNK + k * 16, 16)]
            src_u[b, pl.ds(k * 16, 16)] = lax.shift_right_logical(w, 16)
            dst_u[b, pl.ds(k * 16, 16)] = lax.bitwise_and(w, 0xFFFF)

    for b in range(NBUF - 1):
        unpack(b, b)
        pltpu.async_copy(h_hbm.at[src_u.at[b]], rows_v.at[b], sems.at[b])

    pltpu.sync_copy(h_hbm.at[pl.ds(r0, RPT)], agg_sp.at[pl.ds(r0, RPT)])

    @pl.when(s == NS - 1)
    def _():
        pltpu.sync_copy(h_hbm.at[pl.ds(NS * RPT, RREM)],
                        agg_sp.at[pl.ds(NS * RPT, RREM)])

    plsc.subcore_barrier()


    def group(g, _):
        for b in range(NBUF):
            ch = g * NBUF + b
            b2 = (b + NBUF - 1) % NBUF

            @pl.when(ch < NGRP)
            def _():
                pltpu.make_async_copy(h_hbm.at[src_u.at[b]], rows_v.at[b],
                                      sems.at[b]).wait()

            @pl.when(ch + NBUF - 1 < NGRP)
            def _():
                unpack(ch + NBUF - 1, b2)
                pltpu.async_copy(h_hbm.at[src_u.at[b2]], rows_v.at[b2],
                                 sems.at[b2])

            @pl.when(ch < NGRP)
            def _():
                pltpu.sync_copy(rows_v.at[b], agg_sp.at[dst_u.at[b]],
                                add=True)

        return 0

    lax.fori_loop(0, pl.cdiv(NGRP, NBUF), group, 0)
    plsc.subcore_barrier()
    pltpu.sync_copy(agg_sp.at[pl.ds(r0, RPT)], out_hbm.at[c, pl.ds(r0, RPT)])

    @pl.when(s == NS - 1)
    def _():
        pltpu.sync_copy(agg_sp.at[pl.ds(NS * RPT, RREM)],
                        out_hbm.at[c, pl.ds(NS * RPT, RREM)])


def _dense1_body(x_ref, w_ref, hist_ref, h_ref, dinv_ref):
    deg = 1.0 + hist_ref[:, 0:1] + hist_ref[:, 1:2]
    dinv = lax.rsqrt(deg)
    h = jnp.dot(x_ref[...], w_ref[...], preferred_element_type=jnp.float32)
    h_ref[...] = h * dinv
    dinv_ref[...] = dinv


def _mid_body(agg_ref, h_ref, dinv_ref, b_ref, g_ref, be_ref, w_ref, out_ref):
    dinv = dinv_ref[...]
    t = dinv * (agg_ref[0] + agg_ref[1] - h_ref[...]) + b_ref[...]
    t = g_ref[...] * (t * BN_SCALE) + be_ref[...]
    t = jnp.maximum(t, 0.0)
    out_ref[...] = dinv * jnp.dot(t, w_ref[...],
                                  preferred_element_type=jnp.float32)


def _fin_body(agg_ref, h_ref, dinv_ref, b_ref, out_ref):
    out_ref[...] = (dinv_ref[...] * (agg_ref[0] + agg_ref[1] - h_ref[...])
                    + b_ref[...])


RB = 2000
_GRID = N // RB

_dense1 = pl.pallas_call(
    _dense1_body,
    grid=(_GRID,),
    in_specs=[
        pl.BlockSpec((RB, D), lambda i: (i, 0)),
        pl.BlockSpec((D, D), lambda i: (0, 0)),
        pl.BlockSpec((RB, 2), lambda i: (i, 0)),
    ],
    out_specs=(pl.BlockSpec((RB, D), lambda i: (i, 0)),
               pl.BlockSpec((RB, 1), lambda i: (i, 0))),
    out_shape=(jax.ShapeDtypeStruct((N, D), jnp.float32),
               jax.ShapeDtypeStruct((N, 1), jnp.float32)),
)

_mid = pl.pallas_call(
    _mid_body,
    grid=(_GRID,),
    in_specs=[
        pl.BlockSpec((NC, RB, D), lambda i: (0, i, 0)),
        pl.BlockSpec((RB, D), lambda i: (i, 0)),
        pl.BlockSpec((RB, 1), lambda i: (i, 0)),
        pl.BlockSpec((1, D), lambda i: (0, 0)),
        pl.BlockSpec((1, D), lambda i: (0, 0)),
        pl.BlockSpec((1, D), lambda i: (0, 0)),
        pl.BlockSpec((D, D), lambda i: (0, 0)),
    ],
    out_specs=pl.BlockSpec((RB, D), lambda i: (i, 0)),
    out_shape=jax.ShapeDtypeStruct((N, D), jnp.float32),
)

_fin = pl.pallas_call(
    _fin_body,
    grid=(_GRID,),
    in_specs=[
        pl.BlockSpec((NC, RB, D), lambda i: (0, i, 0)),
        pl.BlockSpec((RB, D), lambda i: (i, 0)),
        pl.BlockSpec((RB, 1), lambda i: (i, 0)),
        pl.BlockSpec((1, D), lambda i: (0, 0)),
    ],
    out_specs=pl.BlockSpec((RB, D), lambda i: (i, 0)),
    out_shape=jax.ShapeDtypeStruct((N, D), jnp.float32),
)


def kernel(x, edge_index, W1, b1, g1, be1, W2, b2, g2, be2, W3, b3):
    ei_flat = edge_index.reshape(2 * E)

    idxp, hist = _deg_sc(ei_flat)
    histT = hist.reshape(NC, N).T
    h1p, dinv = _dense1(x, W1, histT)
    agg1 = _agg_sc(h1p, idxp)
    h2p = _mid(agg1, h1p, dinv, b1.reshape(1, D), g1.reshape(1, D),
               be1.reshape(1, D), W2)
    agg2 = _agg_sc(h2p, idxp)
    h3p = _mid(agg2, h2p, dinv, b2.reshape(1, D), g2.reshape(1, D),
               be2.reshape(1, D), W3)
    agg3 = _agg_sc(h3p, idxp)
    return _fin(agg3, h3p, dinv, b3.reshape(1, D))
